# Initial kernel scaffold; baseline (speedup 1.0000x reference)
import jax
import jax.numpy as jnp
from jax.experimental import pallas as pl


def kernel(x, codebook):
    raise NotImplementedError("write your pallas kernel here")


# speedup vs baseline: 1.0063x; 1.0063x over previous
import functools

import jax
import jax.numpy as jnp
from jax import lax
from jax.experimental import pallas as pl
from jax.experimental.pallas import tpu as pltpu
from jax.experimental.pallas import tpu_sc as plsc

_B, _T, _D = 16, 1024, 256
_K = 1024
_N = _B * _T
_BLK = 512
_NBLK = _N // _BLK
_COMMIT = 0.25


def _dist_argmin_kernel(x_ref, cb_ref, idx_ref, bsum_ref):
    i = pl.program_id(0)
    x = x_ref[...]
    cb = cb_ref[...]
    a2 = jnp.sum(x * x, axis=1, keepdims=True)
    b2 = jnp.sum(cb * cb, axis=1)
    ab = lax.dot_general(x, cb, (((1,), (1,)), ((), ())),
                         preferred_element_type=jnp.float32)
    d = a2 - 2.0 * ab + b2[None, :]
    minval = jnp.min(d, axis=1, keepdims=True)
    iota = lax.broadcasted_iota(jnp.int32, (_BLK, _K), 1)
    idx = jnp.min(jnp.where(d == minval, iota, _K), axis=1)
    idx_ref[...] = idx
    bsum_ref[i] = jnp.sum(minval)


def _dist_argmin(x2, cb):
    return pl.pallas_call(
        _dist_argmin_kernel,
        grid=(_NBLK,),
        in_specs=[
            pl.BlockSpec((_BLK, _D), lambda i: (i, 0)),
            pl.BlockSpec((_K, _D), lambda i: (0, 0)),
        ],
        out_specs=[
            pl.BlockSpec((_BLK,), lambda i: (i,)),
            pl.BlockSpec(memory_space=pltpu.SMEM),
        ],
        out_shape=[
            jax.ShapeDtypeStruct((_N,), jnp.int32),
            jax.ShapeDtypeStruct((_NBLK,), jnp.float32),
        ],
    )(x2, cb)


_NW = 32
_BPW = _N // _NW
_CH = 128
_NCH = _BPW // _CH


def _sc_gather(cb, idx):
    mesh = plsc.VectorSubcoreMesh(core_axis_name="c", subcore_axis_name="s")

    @functools.partial(
        pl.kernel, mesh=mesh,
        out_type=jax.ShapeDtypeStruct((_N, _D), jnp.float32),
        scratch_types=[
            pltpu.VMEM((_BPW,), jnp.int32),
            pltpu.VMEM((_CH, _D), jnp.float32),
            pltpu.SemaphoreType.DMA,
        ],
    )
    def k(table_hbm, idx_hbm, out_hbm, idx_v, rows_v, sem):
        wid = lax.axis_index("s") * 2 + lax.axis_index("c")
        base = wid * _BPW
        pltpu.sync_copy(idx_hbm.at[pl.ds(base, _BPW)], idx_v)
        for c in range(_NCH):
            pltpu.async_copy(
                table_hbm.at[idx_v.at[pl.ds(c * _CH, _CH)]], rows_v, sem
            ).wait()
            pltpu.sync_copy(rows_v, out_hbm.at[pl.ds(base + c * _CH, _CH)])

    return k(cb, idx)


def kernel(x, codebook):
    x2 = x.reshape(_N, _D)
    idx, bsums = _dist_argmin(x2, codebook)
    quantized = _sc_gather(codebook, idx).reshape(_B, _T, _D)
    loss = jnp.sum(bsums) * ((1.0 + _COMMIT) / (_N * _D))
    return quantized, loss, idx.reshape(_B, _T)
